# baseline (speedup 1.0000x reference)
import functools

import jax
import jax.numpy as jnp
from jax import lax
from jax.experimental import pallas as pl
from jax.experimental.pallas import tpu as pltpu
from jax.experimental.pallas import tpu_sc as plsc

EPS = 1e-5
NC = 2
NS = 16
NW = NC * NS
C = 80


def _sc_mesh():
    return plsc.VectorSubcoreMesh(
        core_axis_name="c", subcore_axis_name="s", num_cores=NC, num_subcores=NS
    )


def _make_degree_kernel(N, E):
    CHT = E // NS // C
    CH = CHT // NC
    S_MAIN = (N // NS) // 8 * 8
    TAIL = N - NS * S_MAIN

    @functools.partial(
        pl.kernel,
        out_type=jax.ShapeDtypeStruct((NC * N,), jnp.float32),
        mesh=_sc_mesh(),
        scratch_types=[
            pltpu.VMEM((CH, C), jnp.int32),
            pltpu.VMEM((C,), jnp.float32),
            pltpu.VMEM((S_MAIN,), jnp.float32),
            pltpu.VMEM_SHARED((N,), jnp.float32),
        ],
        compiler_params=pltpu.CompilerParams(use_tc_tiling_on_sc=False),
    )
    def deg_kernel(dst_hbm, zeros_hbm, out_hbm, dst_v, ones_v, zbuf, acc_sh):
        c = lax.axis_index("c")
        s = lax.axis_index("s")
        pltpu.sync_copy(dst_hbm.at[s].at[pl.ds(c * CH, CH)], dst_v)
        for i in range(C // 16):
            ones_v[pl.ds(i * 16, 16)] = jnp.ones((16,), jnp.float32)
        base = s * S_MAIN
        pltpu.sync_copy(zeros_hbm.at[pl.ds(0, S_MAIN)], zbuf)
        pltpu.sync_copy(zbuf, acc_sh.at[pl.ds(base, S_MAIN)])
        if TAIL:
            @pl.when(s == NS - 1)
            def _():
                pltpu.sync_copy(zbuf.at[pl.ds(0, TAIL)],
                                acc_sh.at[pl.ds(NS * S_MAIN, TAIL)])
        plsc.subcore_barrier()

        def body(j, carry):
            pltpu.sync_copy(ones_v, acc_sh.at[dst_v.at[j]], add=True)
            return carry

        lax.fori_loop(0, CH, body, 0)
        plsc.subcore_barrier()
        pltpu.sync_copy(acc_sh.at[pl.ds(base, S_MAIN)], zbuf)
        pltpu.sync_copy(zbuf, out_hbm.at[pl.ds(c * N + base, S_MAIN)])
        if TAIL:
            @pl.when(s == NS - 1)
            def _():
                pltpu.sync_copy(acc_sh.at[pl.ds(NS * S_MAIN, TAIL)],
                                zbuf.at[pl.ds(0, TAIL)])
                pltpu.sync_copy(zbuf.at[pl.ds(0, TAIL)],
                                out_hbm.at[pl.ds(c * N + NS * S_MAIN, TAIL)])

    return deg_kernel


def _make_scatter_kernel(N, Dh, E):
    CH = E // NS // C
    NB = 8
    K = NB // 2
    T = CH // NB
    TL = CH - NB * T
    S2 = (N // NS) // 8 * 8
    TAIL2 = N - NS * S2
    W = S2 // 3
    assert 3 * W == S2 and W % 8 == 0 and 2 * W <= NB * C

    @functools.partial(
        pl.kernel,
        out_type=jax.ShapeDtypeStruct((NC, N, 128), jnp.float32),
        mesh=_sc_mesh(),
        scratch_types=[
            pltpu.VMEM((CH, C), jnp.int32),
            pltpu.VMEM((CH, C), jnp.int32),
            pltpu.VMEM((NB * C, Dh), jnp.float32),
            pltpu.VMEM_SHARED((N, Dh), jnp.float32),
            [pltpu.SemaphoreType.DMA] * NB,
            [pltpu.SemaphoreType.DMA] * NB,
        ],
        compiler_params=pltpu.CompilerParams(use_tc_tiling_on_sc=False),
    )
    def scatter_kernel(y2_hbm, srcx_hbm, dst_hbm, zeros_hbm, out_hbm,
                       src_v, dst_v, rows_v, acc_sh, sem_g, sem_s):
        c = lax.axis_index("c")
        s = lax.axis_index("s")

        def rbuf(b):
            return rows_v.at[pl.ds(b * C, C)]

        pltpu.async_copy(srcx_hbm.at[c, s], src_v, sem_g[0])
        pltpu.async_copy(dst_hbm.at[s], dst_v, sem_g[1])
        zslice = rows_v.at[pl.ds(0, W)]
        pltpu.sync_copy(zeros_hbm, zslice)
        for k in range(3):
            pltpu.sync_copy(zslice, acc_sh.at[pl.ds(s * S2 + k * W, W)])
        if TAIL2:
            @pl.when(s == NS - 1)
            def _():
                pltpu.sync_copy(rows_v.at[pl.ds(0, TAIL2)],
                                acc_sh.at[pl.ds(NS * S2, TAIL2)])
        pltpu.make_async_copy(srcx_hbm.at[c, s], src_v, sem_g[0]).wait()
        pltpu.make_async_copy(dst_hbm.at[s], dst_v, sem_g[1]).wait()
        plsc.subcore_barrier()

        def step(j, b):
            b2 = (b + K) % NB

            @pl.when(j >= K)
            def _():
                pltpu.make_async_copy(
                    rbuf(b2), acc_sh.at[dst_v.at[j - K]],
                    sem_s[b2]).wait()

            @pl.when(j + K < CH)
            def _():
                pltpu.async_copy(y2_hbm.at[src_v.at[j + K]], rbuf(b2),
                                 sem_g[b2])

            pltpu.make_async_copy(
                y2_hbm.at[src_v.at[j]], rbuf(b), sem_g[b]).wait()
            pltpu.async_copy(rbuf(b), acc_sh.at[dst_v.at[j]],
                             sem_s[b], add=True)

        for j0 in range(K):
            pltpu.async_copy(y2_hbm.at[src_v.at[j0]], rbuf(j0), sem_g[j0])

        def body(t, carry):
            for b in range(NB):
                step(t * NB + b, b)
            return carry

        lax.fori_loop(0, T, body, 0)
        for r in range(TL):
            step(NB * T + r, r)
        for j in range(CH - K, CH):
            b = j % NB
            pltpu.make_async_copy(
                rbuf(b), acc_sh.at[dst_v.at[j]], sem_s[b]).wait()
        plsc.subcore_barrier()

        def wslice(k):
            return rows_v.at[pl.ds((k % 2) * W, W)]

        def hbm_dst(off, n):
            return out_hbm.at[c].at[pl.ds(off, n), pl.ds(0, Dh)]

        for k in range(3):
            if k >= 2:
                pltpu.make_async_copy(
                    wslice(k), hbm_dst(s * S2 + (k - 2) * W, W),
                    sem_s[k - 2]).wait()
            pltpu.sync_copy(acc_sh.at[pl.ds(s * S2 + k * W, W)], wslice(k))
            pltpu.async_copy(wslice(k), hbm_dst(s * S2 + k * W, W),
                             sem_s[k % 2])
        for k in range(1, 3):
            pltpu.make_async_copy(
                wslice(k), hbm_dst(s * S2 + k * W, W), sem_s[k % 2]).wait()
        if TAIL2:
            @pl.when(s == NS - 1)
            def _():
                tbuf = rows_v.at[pl.ds(2 * W, TAIL2)]
                pltpu.sync_copy(acc_sh.at[pl.ds(NS * S2, TAIL2)], tbuf)
                pltpu.sync_copy(tbuf, hbm_dst(NS * S2, TAIL2))

    return scatter_kernel


def _first_dense(x, W1, cnt, B):
    N, D = x.shape
    H = W1.shape[1]

    def body(x_ref, w_ref, cnt_ref, y_ref, dinv_ref):
        deg = cnt_ref[0] + cnt_ref[1] + 1.0
        dinv = lax.rsqrt(deg)
        dinv_ref[...] = dinv
        y_ref[...] = jnp.dot(x_ref[...], w_ref[...],
                             preferred_element_type=jnp.float32) * dinv

    return pl.pallas_call(
        body,
        grid=(N // B,),
        in_specs=[
            pl.BlockSpec((B, D), lambda i: (i, 0)),
            pl.BlockSpec((D, H), lambda i: (0, 0)),
            pl.BlockSpec((2, B, 1), lambda i: (0, i, 0)),
        ],
        out_specs=[
            pl.BlockSpec((B, H), lambda i: (i, 0)),
            pl.BlockSpec((B, 1), lambda i: (i, 0)),
        ],
        out_shape=[
            jax.ShapeDtypeStruct((N, H), jnp.float32),
            jax.ShapeDtypeStruct((N, 1), jnp.float32),
        ],
    )(x, W1, cnt)


def _agg_full(p_ref, y_ref, Dh):
    return jnp.concatenate(
        [p_ref[0][:, :Dh], p_ref[1][:, :Dh]], axis=-1) + y_ref[...]


def _mid_dense(p, y, dinv, b, g, bt, W, B):
    N, H = y.shape
    Dh = H // 2

    def body(p_ref, y_ref, dinv_ref, b_ref, g_ref, bt_ref, w_ref, out_ref):
        pre = _agg_full(p_ref, y_ref, Dh) * dinv_ref[...] + b_ref[...]
        m = jnp.mean(pre, axis=-1, keepdims=True)
        d = pre - m
        v = jnp.mean(d * d, axis=-1, keepdims=True)
        h = jnp.maximum(d * lax.rsqrt(v + EPS) * g_ref[...] + bt_ref[...], 0.0)
        out_ref[...] = jnp.dot(h, w_ref[...],
                               preferred_element_type=jnp.float32) * dinv_ref[...]

    return pl.pallas_call(
        body,
        grid=(N // B,),
        in_specs=[
            pl.BlockSpec((2, B, 128), lambda i: (0, i, 0)),
            pl.BlockSpec((B, H), lambda i: (i, 0)),
            pl.BlockSpec((B, 1), lambda i: (i, 0)),
            pl.BlockSpec((1, H), lambda i: (0, 0)),
            pl.BlockSpec((1, H), lambda i: (0, 0)),
            pl.BlockSpec((1, H), lambda i: (0, 0)),
            pl.BlockSpec((H, H), lambda i: (0, 0)),
        ],
        out_specs=pl.BlockSpec((B, H), lambda i: (i, 0)),
        out_shape=jax.ShapeDtypeStruct((N, H), jnp.float32),
    )(p, y, dinv, b, g, bt, W)


def _final_dense(p, y, dinv, b, g, bt, Wl_pad, bl_pad, B):
    N, H = y.shape
    Dh = H // 2

    def body(p_ref, y_ref, dinv_ref, b_ref, g_ref, bt_ref, wl_ref, bl_ref,
             out_ref, acc_ref):
        i = pl.program_id(0)

        @pl.when(i == 0)
        def _():
            acc_ref[...] = jnp.zeros_like(acc_ref)

        pre = _agg_full(p_ref, y_ref, Dh) * dinv_ref[...] + b_ref[...]
        m = jnp.mean(pre, axis=-1, keepdims=True)
        d = pre - m
        v = jnp.mean(d * d, axis=-1, keepdims=True)
        h = jnp.maximum(d * lax.rsqrt(v + EPS) * g_ref[...] + bt_ref[...], 0.0)
        acc_ref[...] += jnp.sum(h, axis=0, keepdims=True)

        @pl.when(i == pl.num_programs(0) - 1)
        def _():
            out_ref[...] = jnp.dot(acc_ref[...] * (1.0 / N), wl_ref[...],
                                   preferred_element_type=jnp.float32) + bl_ref[...]

    return pl.pallas_call(
        body,
        grid=(N // B,),
        in_specs=[
            pl.BlockSpec((2, B, 128), lambda i: (0, i, 0)),
            pl.BlockSpec((B, H), lambda i: (i, 0)),
            pl.BlockSpec((B, 1), lambda i: (i, 0)),
            pl.BlockSpec((1, H), lambda i: (0, 0)),
            pl.BlockSpec((1, H), lambda i: (0, 0)),
            pl.BlockSpec((1, H), lambda i: (0, 0)),
            pl.BlockSpec((H, 128), lambda i: (0, 0)),
            pl.BlockSpec((1, 128), lambda i: (0, 0)),
        ],
        out_specs=pl.BlockSpec((1, 128), lambda i: (0, 0)),
        out_shape=jax.ShapeDtypeStruct((1, 128), jnp.float32),
        scratch_shapes=[pltpu.VMEM((1, 128), jnp.float32)],
    )(p, y, dinv, b, g, bt, Wl_pad, bl_pad)


def kernel(x, edge_index, W1, b1, W2, b2, W3, b3,
           g1, bt1, g2, bt2, g3, bt3, Wl, bl):
    N, D = x.shape
    H = W1.shape[1]
    E = edge_index.shape[1]
    OUT = Wl.shape[1]
    Dh = H // 2
    CHT = E // NS // C
    assert NS * CHT * C == E

    src2 = 2 * edge_index[0]
    srcx = jnp.stack([src2, src2 + 1]).reshape(NC, NS, CHT, C)
    dst16 = edge_index[1].reshape(NS, CHT, C)
    zeros1 = jnp.zeros((N,), jnp.float32)
    zeros2 = jnp.zeros(((N // NS) // 8 * 8 // 3, Dh), jnp.float32)

    deg_k = _make_degree_kernel(N, E)
    scat_k = _make_scatter_kernel(N, Dh, E)

    cnt = deg_k(dst16, zeros1).reshape(NC, N, 1)
    B = 2000
    y1, dinv = _first_dense(x, W1, cnt, B)

    b1r, g1r, bt1r = b1.reshape(1, H), g1.reshape(1, H), bt1.reshape(1, H)
    b2r, g2r, bt2r = b2.reshape(1, H), g2.reshape(1, H), bt2.reshape(1, H)
    b3r, g3r, bt3r = b3.reshape(1, H), g3.reshape(1, H), bt3.reshape(1, H)
    Wl_pad = jnp.zeros((H, 128), jnp.float32).at[:, :OUT].set(Wl)
    bl_pad = jnp.zeros((1, 128), jnp.float32).at[0, :OUT].set(bl)

    p = scat_k(y1.reshape(NC * N, Dh), srcx, dst16, zeros2)
    y2 = _mid_dense(p, y1, dinv, b1r, g1r, bt1r, W2, B)
    p = scat_k(y2.reshape(NC * N, Dh), srcx, dst16, zeros2)
    y3 = _mid_dense(p, y2, dinv, b2r, g2r, bt2r, W3, B)
    p = scat_k(y3.reshape(NC * N, Dh), srcx, dst16, zeros2)
    res = _final_dense(p, y3, dinv, b3r, g3r, bt3r, Wl_pad, bl_pad, B)
    return res[0, :OUT]


# speedup vs baseline: 1.0547x; 1.0547x over previous
import functools

import jax
import jax.numpy as jnp
from jax import lax
from jax.experimental import pallas as pl
from jax.experimental.pallas import tpu as pltpu
from jax.experimental.pallas import tpu_sc as plsc

EPS = 1e-5
NC = 2
NS = 16
NW = NC * NS
C = 80


def _sc_mesh():
    return plsc.VectorSubcoreMesh(
        core_axis_name="c", subcore_axis_name="s", num_cores=NC, num_subcores=NS
    )


def _make_degree_kernel(N, E):
    CHT = E // NS // C
    CH = CHT // NC
    S_MAIN = (N // NS) // 8 * 8
    TAIL = N - NS * S_MAIN

    @functools.partial(
        pl.kernel,
        out_type=jax.ShapeDtypeStruct((NC * N,), jnp.float32),
        mesh=_sc_mesh(),
        scratch_types=[
            pltpu.VMEM((CH, C), jnp.int32),
            pltpu.VMEM((C,), jnp.float32),
            pltpu.VMEM((S_MAIN,), jnp.float32),
            pltpu.VMEM_SHARED((N,), jnp.float32),
        ],
        compiler_params=pltpu.CompilerParams(use_tc_tiling_on_sc=False),
    )
    def deg_kernel(dst_hbm, zeros_hbm, out_hbm, dst_v, ones_v, zbuf, acc_sh):
        c = lax.axis_index("c")
        s = lax.axis_index("s")
        pltpu.sync_copy(dst_hbm.at[s].at[pl.ds(c * CH, CH)], dst_v)
        for i in range(C // 16):
            ones_v[pl.ds(i * 16, 16)] = jnp.ones((16,), jnp.float32)
        base = s * S_MAIN
        pltpu.sync_copy(zeros_hbm.at[pl.ds(0, S_MAIN)], zbuf)
        pltpu.sync_copy(zbuf, acc_sh.at[pl.ds(base, S_MAIN)])
        if TAIL:
            @pl.when(s == NS - 1)
            def _():
                pltpu.sync_copy(zbuf.at[pl.ds(0, TAIL)],
                                acc_sh.at[pl.ds(NS * S_MAIN, TAIL)])
        plsc.subcore_barrier()

        def body(j, carry):
            pltpu.sync_copy(ones_v, acc_sh.at[dst_v.at[j]], add=True)
            return carry

        lax.fori_loop(0, CH, body, 0)
        plsc.subcore_barrier()
        pltpu.sync_copy(acc_sh.at[pl.ds(base, S_MAIN)], zbuf)
        pltpu.sync_copy(zbuf, out_hbm.at[pl.ds(c * N + base, S_MAIN)])
        if TAIL:
            @pl.when(s == NS - 1)
            def _():
                pltpu.sync_copy(acc_sh.at[pl.ds(NS * S_MAIN, TAIL)],
                                zbuf.at[pl.ds(0, TAIL)])
                pltpu.sync_copy(zbuf.at[pl.ds(0, TAIL)],
                                out_hbm.at[pl.ds(c * N + NS * S_MAIN, TAIL)])

    return deg_kernel


def _make_scatter_kernel(N, Dh, E):
    CH = E // NS // C
    NB = 6
    K = NB // 2
    T = CH // NB
    TL = CH - NB * T
    S2 = (N // NS) // 8 * 8
    TAIL2 = N - NS * S2
    W = S2 // 3
    assert 3 * W == S2 and W % 8 == 0 and 2 * W <= NB * C

    @functools.partial(
        pl.kernel,
        out_type=jax.ShapeDtypeStruct((NC, N, 128), jnp.float32),
        mesh=_sc_mesh(),
        scratch_types=[
            pltpu.VMEM((CH, C), jnp.int32),
            pltpu.VMEM((CH, C), jnp.int32),
            pltpu.VMEM((NB * C, Dh), jnp.float32),
            pltpu.VMEM_SHARED((N, Dh), jnp.float32),
            [pltpu.SemaphoreType.DMA] * NB,
            [pltpu.SemaphoreType.DMA] * NB,
        ],
        compiler_params=pltpu.CompilerParams(use_tc_tiling_on_sc=False),
    )
    def scatter_kernel(y2_hbm, srcx_hbm, dst_hbm, zeros_hbm, out_hbm,
                       src_v, dst_v, rows_v, acc_sh, sem_g, sem_s):
        c = lax.axis_index("c")
        s = lax.axis_index("s")

        def rbuf(b):
            return rows_v.at[pl.ds(b * C, C)]

        pltpu.async_copy(srcx_hbm.at[c, s], src_v, sem_g[0])
        pltpu.async_copy(dst_hbm.at[s], dst_v, sem_g[1])
        zslice = rows_v.at[pl.ds(0, W)]
        pltpu.sync_copy(zeros_hbm, zslice)
        for k in range(3):
            pltpu.sync_copy(zslice, acc_sh.at[pl.ds(s * S2 + k * W, W)])
        if TAIL2:
            @pl.when(s == NS - 1)
            def _():
                pltpu.sync_copy(rows_v.at[pl.ds(0, TAIL2)],
                                acc_sh.at[pl.ds(NS * S2, TAIL2)])
        pltpu.make_async_copy(srcx_hbm.at[c, s], src_v, sem_g[0]).wait()
        pltpu.make_async_copy(dst_hbm.at[s], dst_v, sem_g[1]).wait()
        plsc.subcore_barrier()

        def step(j, b):
            b2 = (b + K) % NB

            @pl.when(j >= K)
            def _():
                pltpu.make_async_copy(
                    rbuf(b2), acc_sh.at[dst_v.at[j - K]],
                    sem_s[b2]).wait()

            @pl.when(j + K < CH)
            def _():
                pltpu.async_copy(y2_hbm.at[src_v.at[j + K]], rbuf(b2),
                                 sem_g[b2])

            pltpu.make_async_copy(
                y2_hbm.at[src_v.at[j]], rbuf(b), sem_g[b]).wait()
            pltpu.async_copy(rbuf(b), acc_sh.at[dst_v.at[j]],
                             sem_s[b], add=True)

        for j0 in range(K):
            pltpu.async_copy(y2_hbm.at[src_v.at[j0]], rbuf(j0), sem_g[j0])

        def body(t, carry):
            for b in range(NB):
                step(t * NB + b, b)
            return carry

        lax.fori_loop(0, T, body, 0)
        for r in range(TL):
            step(NB * T + r, r)
        for j in range(CH - K, CH):
            b = j % NB
            pltpu.make_async_copy(
                rbuf(b), acc_sh.at[dst_v.at[j]], sem_s[b]).wait()
        plsc.subcore_barrier()

        def wslice(k):
            return rows_v.at[pl.ds((k % 2) * W, W)]

        def hbm_dst(off, n):
            return out_hbm.at[c].at[pl.ds(off, n), pl.ds(0, Dh)]

        for k in range(3):
            if k >= 2:
                pltpu.make_async_copy(
                    wslice(k), hbm_dst(s * S2 + (k - 2) * W, W),
                    sem_s[k - 2]).wait()
            pltpu.sync_copy(acc_sh.at[pl.ds(s * S2 + k * W, W)], wslice(k))
            pltpu.async_copy(wslice(k), hbm_dst(s * S2 + k * W, W),
                             sem_s[k % 2])
        for k in range(1, 3):
            pltpu.make_async_copy(
                wslice(k), hbm_dst(s * S2 + k * W, W), sem_s[k % 2]).wait()
        if TAIL2:
            @pl.when(s == NS - 1)
            def _():
                tbuf = rows_v.at[pl.ds(2 * W, TAIL2)]
                pltpu.sync_copy(acc_sh.at[pl.ds(NS * S2, TAIL2)], tbuf)
                pltpu.sync_copy(tbuf, hbm_dst(NS * S2, TAIL2))

    return scatter_kernel


def _first_dense(x, W1, cnt, B):
    N, D = x.shape
    H = W1.shape[1]

    def body(x_ref, w_ref, cnt_ref, y_ref, dinv_ref):
        deg = cnt_ref[0] + cnt_ref[1] + 1.0
        dinv = lax.rsqrt(deg)
        dinv_ref[...] = dinv
        y_ref[...] = jnp.dot(x_ref[...], w_ref[...],
                             preferred_element_type=jnp.float32) * dinv

    return pl.pallas_call(
        body,
        grid=(N // B,),
        in_specs=[
            pl.BlockSpec((B, D), lambda i: (i, 0)),
            pl.BlockSpec((D, H), lambda i: (0, 0)),
            pl.BlockSpec((2, B, 1), lambda i: (0, i, 0)),
        ],
        out_specs=[
            pl.BlockSpec((B, H), lambda i: (i, 0)),
            pl.BlockSpec((B, 1), lambda i: (i, 0)),
        ],
        out_shape=[
            jax.ShapeDtypeStruct((N, H), jnp.float32),
            jax.ShapeDtypeStruct((N, 1), jnp.float32),
        ],
    )(x, W1, cnt)


def _agg_full(p_ref, y_ref, Dh):
    return jnp.concatenate(
        [p_ref[0][:, :Dh], p_ref[1][:, :Dh]], axis=-1) + y_ref[...]


def _mid_dense(p, y, dinv, b, g, bt, W, B):
    N, H = y.shape
    Dh = H // 2

    def body(p_ref, y_ref, dinv_ref, b_ref, g_ref, bt_ref, w_ref, out_ref):
        pre = _agg_full(p_ref, y_ref, Dh) * dinv_ref[...] + b_ref[...]
        m = jnp.mean(pre, axis=-1, keepdims=True)
        d = pre - m
        v = jnp.mean(d * d, axis=-1, keepdims=True)
        h = jnp.maximum(d * lax.rsqrt(v + EPS) * g_ref[...] + bt_ref[...], 0.0)
        out_ref[...] = jnp.dot(h, w_ref[...],
                               preferred_element_type=jnp.float32) * dinv_ref[...]

    return pl.pallas_call(
        body,
        grid=(N // B,),
        in_specs=[
            pl.BlockSpec((2, B, 128), lambda i: (0, i, 0)),
            pl.BlockSpec((B, H), lambda i: (i, 0)),
            pl.BlockSpec((B, 1), lambda i: (i, 0)),
            pl.BlockSpec((1, H), lambda i: (0, 0)),
            pl.BlockSpec((1, H), lambda i: (0, 0)),
            pl.BlockSpec((1, H), lambda i: (0, 0)),
            pl.BlockSpec((H, H), lambda i: (0, 0)),
        ],
        out_specs=pl.BlockSpec((B, H), lambda i: (i, 0)),
        out_shape=jax.ShapeDtypeStruct((N, H), jnp.float32),
    )(p, y, dinv, b, g, bt, W)


def _final_dense(p, y, dinv, b, g, bt, Wl_pad, bl_pad, B):
    N, H = y.shape
    Dh = H // 2

    def body(p_ref, y_ref, dinv_ref, b_ref, g_ref, bt_ref, wl_ref, bl_ref,
             out_ref, acc_ref):
        i = pl.program_id(0)

        @pl.when(i == 0)
        def _():
            acc_ref[...] = jnp.zeros_like(acc_ref)

        pre = _agg_full(p_ref, y_ref, Dh) * dinv_ref[...] + b_ref[...]
        m = jnp.mean(pre, axis=-1, keepdims=True)
        d = pre - m
        v = jnp.mean(d * d, axis=-1, keepdims=True)
        h = jnp.maximum(d * lax.rsqrt(v + EPS) * g_ref[...] + bt_ref[...], 0.0)
        acc_ref[...] += jnp.sum(h, axis=0, keepdims=True)

        @pl.when(i == pl.num_programs(0) - 1)
        def _():
            out_ref[...] = jnp.dot(acc_ref[...] * (1.0 / N), wl_ref[...],
                                   preferred_element_type=jnp.float32) + bl_ref[...]

    return pl.pallas_call(
        body,
        grid=(N // B,),
        in_specs=[
            pl.BlockSpec((2, B, 128), lambda i: (0, i, 0)),
            pl.BlockSpec((B, H), lambda i: (i, 0)),
            pl.BlockSpec((B, 1), lambda i: (i, 0)),
            pl.BlockSpec((1, H), lambda i: (0, 0)),
            pl.BlockSpec((1, H), lambda i: (0, 0)),
            pl.BlockSpec((1, H), lambda i: (0, 0)),
            pl.BlockSpec((H, 128), lambda i: (0, 0)),
            pl.BlockSpec((1, 128), lambda i: (0, 0)),
        ],
        out_specs=pl.BlockSpec((1, 128), lambda i: (0, 0)),
        out_shape=jax.ShapeDtypeStruct((1, 128), jnp.float32),
        scratch_shapes=[pltpu.VMEM((1, 128), jnp.float32)],
    )(p, y, dinv, b, g, bt, Wl_pad, bl_pad)


def kernel(x, edge_index, W1, b1, W2, b2, W3, b3,
           g1, bt1, g2, bt2, g3, bt3, Wl, bl):
    N, D = x.shape
    H = W1.shape[1]
    E = edge_index.shape[1]
    OUT = Wl.shape[1]
    Dh = H // 2
    CHT = E // NS // C
    assert NS * CHT * C == E

    src2 = 2 * edge_index[0]
    srcx = jnp.stack([src2, src2 + 1]).reshape(NC, NS, CHT, C)
    dst16 = edge_index[1].reshape(NS, CHT, C)
    zeros1 = jnp.zeros((N,), jnp.float32)
    zeros2 = jnp.zeros(((N // NS) // 8 * 8 // 3, Dh), jnp.float32)

    deg_k = _make_degree_kernel(N, E)
    scat_k = _make_scatter_kernel(N, Dh, E)

    cnt = deg_k(dst16, zeros1).reshape(NC, N, 1)
    B = 2000
    y1, dinv = _first_dense(x, W1, cnt, B)

    b1r, g1r, bt1r = b1.reshape(1, H), g1.reshape(1, H), bt1.reshape(1, H)
    b2r, g2r, bt2r = b2.reshape(1, H), g2.reshape(1, H), bt2.reshape(1, H)
    b3r, g3r, bt3r = b3.reshape(1, H), g3.reshape(1, H), bt3.reshape(1, H)
    Wl_pad = jnp.zeros((H, 128), jnp.float32).at[:, :OUT].set(Wl)
    bl_pad = jnp.zeros((1, 128), jnp.float32).at[0, :OUT].set(bl)

    p = scat_k(y1.reshape(NC * N, Dh), srcx, dst16, zeros2)
    y2 = _mid_dense(p, y1, dinv, b1r, g1r, bt1r, W2, B)
    p = scat_k(y2.reshape(NC * N, Dh), srcx, dst16, zeros2)
    y3 = _mid_dense(p, y2, dinv, b2r, g2r, bt2r, W3, B)
    p = scat_k(y3.reshape(NC * N, Dh), srcx, dst16, zeros2)
    res = _final_dense(p, y3, dinv, b3r, g3r, bt3r, Wl_pad, bl_pad, B)
    return res[0, :OUT]
